# baseline (speedup 1.0000x reference)
import functools
import jax
import jax.numpy as jnp
from jax import lax
from jax.experimental import pallas as pl
from jax.experimental.pallas import tpu as pltpu
from jax.experimental.pallas import tpu_sc as plsc

_T = 1.3
_TAU = 0.9
_KREG = 5
_LAMDA = 0.01

_B, _V = 128, 100000
_CH = 20000
_NCH = _V // _CH
_VECS = _CH // 16
_UNROLL = 5
_VITER = _VECS // _UNROLL
_NB = 1024
_NBV = _NB // 16
_ROWS_PER_TILE = 4

_mesh = plsc.VectorSubcoreMesh(core_axis_name="c", subcore_axis_name="s")


def _splat_f(x):
    return jnp.full((16,), x, jnp.float32)


@functools.partial(
    pl.kernel,
    mesh=_mesh,
    out_type=jax.ShapeDtypeStruct((_B * _V,), jnp.float32),
    compiler_params=pltpu.CompilerParams(needs_layout_passes=False),
    scratch_types=[
        pltpu.VMEM((_CH,), jnp.float32),
        pltpu.VMEM((_CH,), jnp.float32),
        pltpu.VMEM((_CH,), jnp.float32),
        pltpu.VMEM((_CH,), jnp.float32),
        pltpu.VMEM((_NB,), jnp.float32),
        pltpu.VMEM((_NB,), jnp.float32),
        pltpu.SemaphoreType.DMA,
        pltpu.SemaphoreType.DMA,
    ],
)
def _sc_raps(x_hbm, o_hbm, xa, xb, oa, ob, cnt_h, sum_h, sem_r, sem_w):
    wid = lax.axis_index("s") * 2 + lax.axis_index("c")
    xbufs = (xa, xb)
    obufs = (oa, ob)

    def rd(src, row, c, buf):
        return pltpu.async_copy(src.at[pl.ds(row * _V + c * _CH, _CH)], buf,
                                sem_r)

    def wr(buf, row, c):
        return pltpu.async_copy(buf, o_hbm.at[pl.ds(row * _V + c * _CH, _CH)],
                                sem_w)

    def zero_hist():
        def zh(i, c):
            z = jnp.zeros((16,), jnp.float32)
            cnt_h[pl.ds(i * 16, 16)] = z
            sum_h[pl.ds(i * 16, 16)] = z
            return c
        lax.fori_loop(0, _NBV, zh, 0)

    def scan_level(n_prevv, s_prevv, invZv):
        def tot(i, acc):
            ac, as_ = acc
            return (ac + cnt_h[pl.ds(i * 16, 16)], as_ + sum_h[pl.ds(i * 16, 16)])
        tcv, tsv = lax.fori_loop(0, _NBV, tot, (jnp.zeros((16,), jnp.float32),
                                                jnp.zeros((16,), jnp.float32)))
        cnt_totv = _splat_f(jnp.sum(tcv))
        sum_totv = _splat_f(jnp.sum(tsv))

        def scan(i, carry):
            Pcv, Psv, nbv, n_incv, s_incv = carry
            c = cnt_h[pl.ds(i * 16, 16)]
            s = sum_h[pl.ds(i * 16, 16)]
            pc = plsc.cumsum(c) + Pcv
            ps = plsc.cumsum(s) + Psv
            suffix_n = (n_prevv + cnt_totv) - (pc - c)
            suffix_s = (s_prevv + sum_totv) - (ps - s)
            G = suffix_s * invZv + _LAMDA * jnp.maximum(0.0, suffix_n - _KREG)
            t = G > _TAU
            nbv = nbv + jnp.where(t, 1, 0)
            n_incv = n_incv + jnp.where(t, c, 0.0)
            s_incv = s_incv + jnp.where(t, s, 0.0)
            Pcv = Pcv + _splat_f(jnp.sum(c))
            Psv = Psv + _splat_f(jnp.sum(s))
            return (Pcv, Psv, nbv, n_incv, s_incv)

        z = jnp.zeros((16,), jnp.float32)
        zi = jnp.zeros((16,), jnp.int32)
        Pcv, Psv, nbv, n_incv, s_incv = lax.fori_loop(0, _NBV, scan, (z, z, zi, z, z))
        b = jnp.maximum(jnp.sum(nbv) - 1, 0)
        n_above = n_prevv + cnt_totv - _splat_f(jnp.sum(n_incv))
        s_above = s_prevv + sum_totv - _splat_f(jnp.sum(s_incv))
        cnt_b = plsc.load_gather(cnt_h, [jnp.full((16,), b, jnp.int32)])
        return b, n_above, s_above, cnt_b

    def refine_pass(row, key_shift, match_shift, match_val):
        h = rd(o_hbm, row, 0, xbufs[0])
        for c in range(_NCH):
            h.wait()
            if c + 1 < _NCH:
                h = rd(o_hbm, row, c + 1, xbufs[(c + 1) % 2])
            buf = xbufs[c % 2]

            def vec(i, acc):
                ones = jnp.full((16,), 1.0, jnp.float32)
                for j in range(_UNROLL):
                    off = (i * _UNROLL + j) * 16
                    e = buf[pl.ds(off, 16)]
                    p = lax.bitcast_convert_type(e, jnp.int32)
                    k = lax.shift_right_logical(p, key_shift) & (_NB - 1)
                    msk = lax.shift_right_logical(p, match_shift) == match_val
                    plsc.addupdate_scatter(cnt_h, [k], ones, mask=msk)
                    plsc.addupdate_scatter(sum_h, [k], e, mask=msk)
                return acc

            lax.fori_loop(0, _VITER, vec, 0)

    def do_row(r, carry):
        row = wid * _ROWS_PER_TILE + r

        h = rd(x_hbm, row, 0, xbufs[0])
        mxv = jnp.full((16,), -jnp.inf, jnp.float32)
        for c in range(_NCH):
            h.wait()
            if c + 1 < _NCH:
                h = rd(x_hbm, row, c + 1, xbufs[(c + 1) % 2])
            buf = xbufs[c % 2]

            def vec_max(i, mx):
                for j in range(_UNROLL):
                    v = buf[pl.ds((i * _UNROLL + j) * 16, 16)]
                    mx = jnp.maximum(mx, v * (1.0 / _T))
                return mx

            mxv = lax.fori_loop(0, _VITER, vec_max, mxv)
        mv = _splat_f(jnp.max(mxv))

        zero_hist()
        h = rd(x_hbm, row, 0, xbufs[0])
        whs = [None] * _NCH
        accv = jnp.zeros((16,), jnp.float32)
        for c in range(_NCH):
            h.wait()
            if c + 1 < _NCH:
                h = rd(x_hbm, row, c + 1, xbufs[(c + 1) % 2])
            if c >= 2:
                whs[c - 2].wait()
            buf = xbufs[c % 2]
            out = obufs[c % 2]

            def vec1(i, acc):
                ones = jnp.full((16,), 1.0, jnp.float32)
                for j in range(_UNROLL):
                    off = (i * _UNROLL + j) * 16
                    v = buf[pl.ds(off, 16)]
                    e = jnp.exp(v * (1.0 / _T) - mv)
                    out[pl.ds(off, 16)] = e
                    p = lax.bitcast_convert_type(e, jnp.int32)
                    k = lax.shift_right_logical(p, 20) & (_NB - 1)
                    plsc.addupdate_scatter(cnt_h, [k], ones)
                    plsc.addupdate_scatter(sum_h, [k], e)
                    acc = acc + e
                return acc

            accv = lax.fori_loop(0, _VITER, vec1, accv)
            whs[c] = wr(out, row, c)
        whs[_NCH - 2].wait()
        whs[_NCH - 1].wait()

        Zv = _splat_f(jnp.sum(accv))
        invZv = 1.0 / Zv
        zero16 = jnp.zeros((16,), jnp.float32)
        b1, n_ab, s_ab, cb1 = scan_level(zero16, zero16, invZv)
        Mi = lax.convert_element_type(jnp.max(cb1), jnp.int32)

        h = rd(o_hbm, row, 0, xbufs[0])
        gcv = jnp.zeros((16,), jnp.float32)
        for c in range(_NCH):
            h.wait()
            if c + 1 < _NCH:
                h = rd(o_hbm, row, c + 1, xbufs[(c + 1) % 2])
            buf = xbufs[c % 2]

            def vec_g(i, gc):
                for j in range(_UNROLL):
                    off = (i * _UNROLL + j) * 16
                    e = buf[pl.ds(off, 16)]
                    p = lax.bitcast_convert_type(e, jnp.int32)
                    m = lax.shift_right_logical(p, 20) == b1
                    mf = jnp.where(m, 1.0, 0.0)
                    pos = plsc.cumsum(mf) - mf + gc
                    idx = pos.astype(jnp.int32)
                    ok = m & (idx < _CH)
                    plsc.store_scatter(oa, [idx], e, mask=ok)
                    gc = gc + _splat_f(jnp.sum(mf))
                return gc

            gcv = lax.fori_loop(0, _VITER, vec_g, gcv)

        def local_hist(key_shift, match_shift, match_val):
            nv = (Mi + 15) // 16

            def vec(i, c):
                base = i * 16
                e = oa[pl.ds(base, 16)]
                valid = (lax.iota(jnp.int32, 16) + base) < Mi
                p = lax.bitcast_convert_type(e, jnp.int32)
                k = lax.shift_right_logical(p, key_shift) & (_NB - 1)
                if match_shift >= 0:
                    valid = valid & (
                        (lax.shift_right_logical(p, match_shift) & (_NB - 1))
                        == match_val)
                ones = jnp.full((16,), 1.0, jnp.float32)
                plsc.addupdate_scatter(cnt_h, [k], ones, mask=valid)
                plsc.addupdate_scatter(sum_h, [k], e, mask=valid)
                return c

            lax.fori_loop(0, nv, vec, 0)

        def local_branch(_, carry):
            _b, n0, s0, _m = carry
            zero_hist()
            local_hist(10, -1, 0)
            b2, n1, s1, _c2 = scan_level(n0, s0, invZv)
            zero_hist()
            local_hist(0, 10, b2)
            b3, n2, s2, m = scan_level(n1, s1, invZv)
            return (b2 * _NB + b3, n2, s2, m)

        def stream_branch(_, carry):
            _b, n0, s0, _m = carry
            zero_hist()
            refine_pass(row, 10, 20, b1)
            b2, n1, s1, _c2 = scan_level(n0, s0, invZv)
            zero_hist()
            refine_pass(row, 0, 10, b1 * _NB + b2)
            b3, n2, s2, m = scan_level(n1, s1, invZv)
            return (b2 * _NB + b3, n2, s2, m)

        t_local = jnp.where(Mi <= _CH, 1, 0)
        ref0 = (0 * t_local, n_ab, s_ab, jnp.zeros((16,), jnp.float32))
        ref1 = lax.fori_loop(0, t_local, local_branch, ref0)
        ref2 = lax.fori_loop(0, 1 - t_local, stream_branch, ref1)
        b23, n_ab, s_ab, m_eqv = ref2

        vc_pat = b1 * (_NB * _NB) + b23
        vcv = lax.bitcast_convert_type(jnp.full((16,), vc_pat, jnp.int32),
                                       jnp.float32)

        def en_body(i, acc):
            rr = (lax.iota(jnp.int32, 16) + (i * 16 + 1)).astype(jnp.float32)
            f_r = (s_ab + rr * vcv) * invZv + _LAMDA * jnp.maximum(
                0.0, n_ab + rr - _KREG)
            ok = (f_r <= _TAU) & (rr <= m_eqv)
            return acc + jnp.where(ok, 1.0, 0.0)

        okv = lax.fori_loop(0, 8, en_body, jnp.zeros((16,), jnp.float32))
        e_needv = _splat_f(jnp.sum(okv) + 1.0)

        h = rd(o_hbm, row, 0, xbufs[0])
        whs4 = [None] * _NCH
        cntv = jnp.zeros((16,), jnp.float32)
        for c in range(_NCH):
            h.wait()
            if c + 1 < _NCH:
                h = rd(o_hbm, row, c + 1, xbufs[(c + 1) % 2])
            if c >= 2:
                whs4[c - 2].wait()
            buf = xbufs[c % 2]
            out = obufs[c % 2]

            def vec_p4(i, cntv):
                for j in range(_UNROLL):
                    off = (i * _UNROLL + j) * 16
                    e = buf[pl.ds(off, 16)]
                    eq = e == vcv
                    eqf = jnp.where(eq, 1.0, 0.0)
                    pr = plsc.cumsum(eqf) + cntv
                    sel = (e > vcv) | (eq & (pr <= e_needv))
                    out[pl.ds(off, 16)] = jnp.where(sel, e * invZv, 0.0)
                    cntv = cntv + _splat_f(jnp.sum(eqf))
                return cntv

            cntv = lax.fori_loop(0, _VITER, vec_p4, cntv)
            whs4[c] = wr(out, row, c)
        whs4[_NCH - 2].wait()
        whs4[_NCH - 1].wait()
        return carry

    lax.fori_loop(0, _ROWS_PER_TILE, do_row, 0)


def kernel(logits):
    B, V = logits.shape
    return _sc_raps(logits.reshape(B * V)).reshape(B, V)


# speedup vs baseline: 1.2029x; 1.2029x over previous
import functools
import jax
import jax.numpy as jnp
from jax import lax
from jax.experimental import pallas as pl
from jax.experimental.pallas import tpu as pltpu
from jax.experimental.pallas import tpu_sc as plsc

_T = 1.3
_TAU = 0.9
_KREG = 5
_LAMDA = 0.01

_B, _V = 128, 100000
_CH = 20000
_NCH = _V // _CH
_VECS = _CH // 16
_UNROLL = 5
_VITER = _VECS // _UNROLL
_NB = 1024
_NBV = _NB // 16
_ROWS_PER_TILE = 4

_mesh = plsc.VectorSubcoreMesh(core_axis_name="c", subcore_axis_name="s")


def _splat_f(x):
    return jnp.full((16,), x, jnp.float32)


@functools.partial(
    pl.kernel,
    mesh=_mesh,
    out_type=jax.ShapeDtypeStruct((_B * _V,), jnp.float32),
    compiler_params=pltpu.CompilerParams(needs_layout_passes=False),
    scratch_types=[
        pltpu.VMEM((_CH,), jnp.float32),
        pltpu.VMEM((_CH,), jnp.float32),
        pltpu.VMEM((_CH,), jnp.float32),
        pltpu.VMEM((_CH,), jnp.float32),
        pltpu.VMEM((_NB,), jnp.float32),
        pltpu.VMEM((_NB,), jnp.float32),
        pltpu.SemaphoreType.DMA,
        pltpu.SemaphoreType.DMA,
    ],
)
def _sc_raps(x_hbm, o_hbm, xa, xb, oa, ob, cnt_h, sum_h, sem_r, sem_w):
    wid = lax.axis_index("s") * 2 + lax.axis_index("c")
    xbufs = (xa, xb)
    obufs = (oa, ob)

    def rd(src, row, c, buf):
        return pltpu.async_copy(src.at[pl.ds(row * _V + c * _CH, _CH)], buf,
                                sem_r)

    def wr(buf, row, c):
        return pltpu.async_copy(buf, o_hbm.at[pl.ds(row * _V + c * _CH, _CH)],
                                sem_w)

    def zero_hist():
        def zh(i, c):
            z = jnp.zeros((16,), jnp.float32)
            cnt_h[pl.ds(i * 16, 16)] = z
            sum_h[pl.ds(i * 16, 16)] = z
            return c
        lax.fori_loop(0, _NBV, zh, 0)

    def scan_level(n_prevv, s_prevv, invZv):
        def tot(i, acc):
            ac, as_ = acc
            return (ac + cnt_h[pl.ds(i * 16, 16)], as_ + sum_h[pl.ds(i * 16, 16)])
        tcv, tsv = lax.fori_loop(0, _NBV, tot, (jnp.zeros((16,), jnp.float32),
                                                jnp.zeros((16,), jnp.float32)))
        cnt_totv = _splat_f(jnp.sum(tcv))
        sum_totv = _splat_f(jnp.sum(tsv))

        def scan(i, carry):
            Pcv, Psv, nbv, n_incv, s_incv = carry
            c = cnt_h[pl.ds(i * 16, 16)]
            s = sum_h[pl.ds(i * 16, 16)]
            pc = plsc.cumsum(c) + Pcv
            ps = plsc.cumsum(s) + Psv
            suffix_n = (n_prevv + cnt_totv) - (pc - c)
            suffix_s = (s_prevv + sum_totv) - (ps - s)
            G = suffix_s * invZv + _LAMDA * jnp.maximum(0.0, suffix_n - _KREG)
            t = G > _TAU
            nbv = nbv + jnp.where(t, 1, 0)
            n_incv = n_incv + jnp.where(t, c, 0.0)
            s_incv = s_incv + jnp.where(t, s, 0.0)
            Pcv = Pcv + _splat_f(jnp.sum(c))
            Psv = Psv + _splat_f(jnp.sum(s))
            return (Pcv, Psv, nbv, n_incv, s_incv)

        z = jnp.zeros((16,), jnp.float32)
        zi = jnp.zeros((16,), jnp.int32)
        Pcv, Psv, nbv, n_incv, s_incv = lax.fori_loop(0, _NBV, scan, (z, z, zi, z, z))
        b = jnp.maximum(jnp.sum(nbv) - 1, 0)
        n_above = n_prevv + cnt_totv - _splat_f(jnp.sum(n_incv))
        s_above = s_prevv + sum_totv - _splat_f(jnp.sum(s_incv))
        cnt_b = plsc.load_gather(cnt_h, [jnp.full((16,), b, jnp.int32)])
        return b, n_above, s_above, cnt_b

    def refine_pass(row, key_shift, match_shift, match_val):
        h = rd(o_hbm, row, 0, xbufs[0])
        for c in range(_NCH):
            h.wait()
            if c + 1 < _NCH:
                h = rd(o_hbm, row, c + 1, xbufs[(c + 1) % 2])
            buf = xbufs[c % 2]

            def vec(i, acc):
                ones = jnp.full((16,), 1.0, jnp.float32)
                for j in range(_UNROLL):
                    off = (i * _UNROLL + j) * 16
                    e = buf[pl.ds(off, 16)]
                    p = lax.bitcast_convert_type(e, jnp.int32)
                    k = lax.shift_right_logical(p, key_shift) & (_NB - 1)
                    msk = lax.shift_right_logical(p, match_shift) == match_val
                    plsc.addupdate_scatter(cnt_h, [k], ones, mask=msk)
                    plsc.addupdate_scatter(sum_h, [k], e, mask=msk)
                return acc

            lax.fori_loop(0, _VITER, vec, 0)

    def do_row(r, carry):
        row = wid * _ROWS_PER_TILE + r

        h = rd(x_hbm, row, 0, xbufs[0])
        mxv = jnp.full((16,), -jnp.inf, jnp.float32)
        for c in range(_NCH):
            h.wait()
            if c + 1 < _NCH:
                h = rd(x_hbm, row, c + 1, xbufs[(c + 1) % 2])
            buf = xbufs[c % 2]

            def vec_max(i, mx):
                for j in range(_UNROLL):
                    v = buf[pl.ds((i * _UNROLL + j) * 16, 16)]
                    mx = jnp.maximum(mx, v * (1.0 / _T))
                return mx

            mxv = lax.fori_loop(0, _VITER, vec_max, mxv)
        mv = _splat_f(jnp.max(mxv))

        zero_hist()
        h = rd(x_hbm, row, 0, xbufs[0])
        whs = [None] * _NCH
        accv = jnp.zeros((16,), jnp.float32)
        for c in range(_NCH):
            h.wait()
            if c + 1 < _NCH:
                h = rd(x_hbm, row, c + 1, xbufs[(c + 1) % 2])
            if c >= 2:
                whs[c - 2].wait()
            buf = xbufs[c % 2]
            out = obufs[c % 2]

            def vec1(i, acc):
                ones = jnp.full((16,), 1.0, jnp.float32)
                for j in range(_UNROLL):
                    off = (i * _UNROLL + j) * 16
                    v = buf[pl.ds(off, 16)]
                    e = jnp.exp(v * (1.0 / _T) - mv)
                    out[pl.ds(off, 16)] = e
                    p = lax.bitcast_convert_type(e, jnp.int32)
                    k = lax.shift_right_logical(p, 20) & (_NB - 1)
                    plsc.addupdate_scatter(cnt_h, [k], ones)
                    plsc.addupdate_scatter(sum_h, [k], e)
                    acc = acc + e
                return acc

            accv = lax.fori_loop(0, _VITER, vec1, accv)
            whs[c] = wr(out, row, c)
        whs[_NCH - 2].wait()
        whs[_NCH - 1].wait()

        Zv = _splat_f(jnp.sum(accv))
        invZv = 1.0 / Zv
        zero16 = jnp.zeros((16,), jnp.float32)
        b1, n_ab, s_ab, cb1 = scan_level(zero16, zero16, invZv)

        _SLOTS = _CH // 16
        nanv = lax.bitcast_convert_type(
            jnp.full((16,), 0x7FC00000, jnp.int32), jnp.float32)
        h = rd(o_hbm, row, 0, xbufs[0])
        gs = 0 * b1
        for c in range(_NCH):
            h.wait()
            if c + 1 < _NCH:
                h = rd(o_hbm, row, c + 1, xbufs[(c + 1) % 2])
            buf = xbufs[c % 2]

            def vec_g(i, gs):
                for j in range(_UNROLL):
                    off = (i * _UNROLL + j) * 16
                    e = buf[pl.ds(off, 16)]
                    p = lax.bitcast_convert_type(e, jnp.int32)
                    m = lax.shift_right_logical(p, 20) == b1
                    mf = jnp.where(m, 1.0, 0.0)
                    hit = jnp.sum(mf) > 0.0
                    slot = jnp.minimum(gs, _SLOTS - 1)
                    oa[pl.ds(slot * 16, 16)] = jnp.where(m, e, nanv)
                    gs = gs + jnp.where(hit, 1, 0)
                return gs

            gs = lax.fori_loop(0, _VITER, vec_g, gs)

        def local_hist(key_shift, match_shift, match_val):
            def vec(i, c):
                e = oa[pl.ds(i * 16, 16)]
                valid = e == e
                p = lax.bitcast_convert_type(e, jnp.int32)
                k = lax.shift_right_logical(p, key_shift) & (_NB - 1)
                if match_shift >= 0:
                    valid = valid & (
                        (lax.shift_right_logical(p, match_shift) & (_NB - 1))
                        == match_val)
                ones = jnp.full((16,), 1.0, jnp.float32)
                plsc.addupdate_scatter(cnt_h, [k], ones, mask=valid)
                plsc.addupdate_scatter(sum_h, [k], e, mask=valid)
                return c

            lax.fori_loop(0, jnp.minimum(gs, _SLOTS), vec, 0)

        def local_branch(_, carry):
            _b, n0, s0, _m = carry
            zero_hist()
            local_hist(10, -1, 0)
            b2, n1, s1, _c2 = scan_level(n0, s0, invZv)
            zero_hist()
            local_hist(0, 10, b2)
            b3, n2, s2, m = scan_level(n1, s1, invZv)
            return (b2 * _NB + b3, n2, s2, m)

        def stream_branch(_, carry):
            _b, n0, s0, _m = carry
            zero_hist()
            refine_pass(row, 10, 20, b1)
            b2, n1, s1, _c2 = scan_level(n0, s0, invZv)
            zero_hist()
            refine_pass(row, 0, 10, b1 * _NB + b2)
            b3, n2, s2, m = scan_level(n1, s1, invZv)
            return (b2 * _NB + b3, n2, s2, m)

        t_local = jnp.where(gs <= _SLOTS, 1, 0)
        ref0 = (0 * t_local, n_ab, s_ab, jnp.zeros((16,), jnp.float32))
        ref1 = lax.fori_loop(0, t_local, local_branch, ref0)
        ref2 = lax.fori_loop(0, 1 - t_local, stream_branch, ref1)
        b23, n_ab, s_ab, m_eqv = ref2

        vc_pat = b1 * (_NB * _NB) + b23
        vcv = lax.bitcast_convert_type(jnp.full((16,), vc_pat, jnp.int32),
                                       jnp.float32)

        def en_body(i, acc):
            rr = (lax.iota(jnp.int32, 16) + (i * 16 + 1)).astype(jnp.float32)
            f_r = (s_ab + rr * vcv) * invZv + _LAMDA * jnp.maximum(
                0.0, n_ab + rr - _KREG)
            ok = (f_r <= _TAU) & (rr <= m_eqv)
            return acc + jnp.where(ok, 1.0, 0.0)

        okv = lax.fori_loop(0, 8, en_body, jnp.zeros((16,), jnp.float32))
        e_needv = _splat_f(jnp.sum(okv) + 1.0)

        tA = jnp.where(jnp.max(e_needv) >= jnp.max(m_eqv), 1, 0)

        def p4_simple(_, cc):
            h = rd(o_hbm, row, 0, xbufs[0])
            whs4 = [None] * _NCH
            for c in range(_NCH):
                h.wait()
                if c + 1 < _NCH:
                    h = rd(o_hbm, row, c + 1, xbufs[(c + 1) % 2])
                if c >= 2:
                    whs4[c - 2].wait()
                buf = xbufs[c % 2]
                out = obufs[c % 2]

                def vec_p4(i, cc2):
                    for j in range(_UNROLL):
                        off = (i * _UNROLL + j) * 16
                        e = buf[pl.ds(off, 16)]
                        out[pl.ds(off, 16)] = jnp.where(e >= vcv, e * invZv, 0.0)
                    return cc2

                lax.fori_loop(0, _VITER, vec_p4, 0)
                whs4[c] = wr(out, row, c)
            whs4[_NCH - 2].wait()
            whs4[_NCH - 1].wait()
            return cc

        def p4_ties(_, cc):
            h = rd(o_hbm, row, 0, xbufs[0])
            whs4 = [None] * _NCH
            cntv = jnp.zeros((16,), jnp.float32)
            for c in range(_NCH):
                h.wait()
                if c + 1 < _NCH:
                    h = rd(o_hbm, row, c + 1, xbufs[(c + 1) % 2])
                if c >= 2:
                    whs4[c - 2].wait()
                buf = xbufs[c % 2]
                out = obufs[c % 2]

                def vec_p4(i, cntv):
                    for j in range(_UNROLL):
                        off = (i * _UNROLL + j) * 16
                        e = buf[pl.ds(off, 16)]
                        eq = e == vcv
                        eqf = jnp.where(eq, 1.0, 0.0)
                        pr = plsc.cumsum(eqf) + cntv
                        sel = (e > vcv) | (eq & (pr <= e_needv))
                        out[pl.ds(off, 16)] = jnp.where(sel, e * invZv, 0.0)
                        cntv = cntv + _splat_f(jnp.sum(eqf))
                    return cntv

                cntv = lax.fori_loop(0, _VITER, vec_p4, cntv)
                whs4[c] = wr(out, row, c)
            whs4[_NCH - 2].wait()
            whs4[_NCH - 1].wait()
            return cc

        lax.fori_loop(0, tA, p4_simple, 0)
        lax.fori_loop(0, 1 - tA, p4_ties, 0)
        return carry

    lax.fori_loop(0, _ROWS_PER_TILE, do_row, 0)


def kernel(logits):
    B, V = logits.shape
    return _sc_raps(logits.reshape(B * V)).reshape(B, V)
